# baseline (speedup 1.0000x reference)
import functools

import jax
import jax.numpy as jnp
from jax import lax
from jax.experimental import pallas as pl
from jax.experimental.pallas import tpu as pltpu
from jax.experimental.pallas import tpu_sc as plsc

NC, NS, L = 2, 16, 16
NW = NC * NS

N = 10000
E = 160000
D = 256

K = 128
EPT_A = E // NW
RBITS = 14
RPT = 320
ACCR = 328
CH = 4000
SEL = 4352

_mesh = plsc.VectorSubcoreMesh(
    core_axis_name="c", subcore_axis_name="s", num_cores=NC, num_subcores=NS
)
_sc_params = pltpu.CompilerParams(needs_layout_passes=False)


_DEG_KW = dict(
    out_type=jax.ShapeDtypeStruct((NW, N), jnp.float32),
    mesh=_mesh,
    scratch_types=[
        pltpu.VMEM((EPT_A,), jnp.int32),
        pltpu.VMEM((N,), jnp.float32),
    ],
    compiler_params=_sc_params,
)


def _deg_body(col_hbm, dego_hbm, col_v, deg_v):
    c = lax.axis_index("c")
    s = lax.axis_index("s")
    wid = s * NC + c
    pltpu.sync_copy(col_hbm.at[pl.ds(wid * EPT_A, EPT_A)], col_v)

    def zero_body(i, carry):
        deg_v[pl.ds(i * L, L)] = jnp.zeros((L,), jnp.float32)
        return carry

    lax.fori_loop(0, N // L, zero_body, 0)

    ones = jnp.ones((L,), jnp.float32)
    nfull = EPT_A // L

    def add_body(i, carry):
        idx = col_v[pl.ds(i * L, L)]
        plsc.addupdate_scatter(deg_v, [idx], ones)
        return carry

    lax.fori_loop(0, nfull, add_body, 0)
    rem = EPT_A - nfull * L
    if rem:
        idx = col_v[pl.ds(EPT_A - L, L)]
        m = lax.iota(jnp.int32, L) >= (L - rem)
        plsc.addupdate_scatter(deg_v, [idx], ones, mask=m)
    pltpu.sync_copy(deg_v, dego_hbm.at[wid])


_deg_kernel = functools.partial(pl.kernel, **_DEG_KW)(_deg_body)


_SCAT_KW = dict(
    out_type=jax.ShapeDtypeStruct((N, D), jnp.float32),
    mesh=_mesh,
    scratch_types=[
        pltpu.VMEM((CH,), jnp.int32),
        pltpu.VMEM((CH,), jnp.int32),
        pltpu.VMEM((SEL,), jnp.int32),
        pltpu.VMEM((K,), jnp.int32),
        pltpu.VMEM((K,), jnp.int32),
        pltpu.VMEM((K, D), jnp.float32),
        pltpu.VMEM((ACCR, D), jnp.float32),
        pltpu.SemaphoreType.DMA,
        pltpu.SemaphoreType.DMA,
        pltpu.SemaphoreType.DMA,
        pltpu.SemaphoreType.DMA,
        pltpu.SemaphoreType.DMA,
    ],
    compiler_params=_sc_params,
)


def _scatter_body(pk_hbm, y_hbm, zin_hbm, out_hbm,
                  pk0, pk1, psel, rowstage, lstage, gbuf, acc,
                  gsem0, gsem1, gsem2, esem0, esem1):
    c = lax.axis_index("c")
    s = lax.axis_index("s")
    wid = s * NC + c
    base = wid * RPT

    for q in range(ACCR // K):
        pltpu.sync_copy(zin_hbm, acc.at[pl.ds(q * K, K)])
    remz = ACCR - (ACCR // K) * K
    pltpu.sync_copy(zin_hbm.at[pl.ds(0, remz)], acc.at[pl.ds((ACCR // K) * K, remz)])

    def flush(j, nothing):
        off = j * K
        for k8 in range(K // L):
            p = psel[pl.ds(off + k8 * L, L)]
            rowstage[pl.ds(k8 * L, L)] = p & ((1 << RBITS) - 1)
            lstage[pl.ds(k8 * L, L)] = lax.shift_right_logical(p, RBITS)
        pltpu.async_copy(y_hbm.at[rowstage], gbuf, gsem0).wait()

        @plsc.parallel_loop(0, K // L, unroll=2)
        def add_group(g):
            dvec = lstage[pl.ds(g * L, L)]
            rbase = g * L
            for i in range(L):
                d = dvec[i]
                for k in range(D // L):
                    plsc.addupdate(
                        acc.at[d, pl.ds(k * L, L)], gbuf[rbase + i, pl.ds(k * L, L)]
                    )

        return nothing

    lim = jnp.uint32(RPT << RBITS)
    boff = base << RBITS

    def scan_buf(buf, cnt):
        def comp_body(i, cnt):
            q = buf[pl.ds(i * L, L)] - boff
            m = plsc.bitcast(q, jnp.uint32) < lim
            plsc.store_compressed(psel.at[pl.ds(cnt, L)], q, mask=m)
            return cnt + jnp.sum(m.astype(jnp.int32))

        cnt = lax.fori_loop(0, CH // L, comp_body, cnt)
        nfl = cnt // K
        lax.fori_loop(0, nfl, flush, 0)
        for k8 in range(K // L):
            v = psel[pl.ds(nfl * K + k8 * L, L)]
            psel[pl.ds(k8 * L, L)] = v
        return cnt - nfl * K

    T = E // CH
    pltpu.async_copy(pk_hbm.at[pl.ds(0, CH)], pk0, esem0)

    def scan_pair(tt, cnt):
        t = 2 * tt
        d1 = pltpu.async_copy(pk_hbm.at[pl.ds((t + 1) * CH, CH)], pk1, esem1)
        pltpu.make_async_copy(pk_hbm.at[pl.ds(t * CH, CH)], pk0, esem0).wait()
        cnt = scan_buf(pk0, cnt)

        @pl.when(t + 2 < T)
        def _prefetch():
            pltpu.async_copy(pk_hbm.at[pl.ds((t + 2) * CH, CH)], pk0, esem0)

        d1.wait()
        cnt = scan_buf(pk1, cnt)
        return cnt

    cnt = lax.fori_loop(0, T // 2, scan_pair, jnp.int32(0))

    dumpv = ((RPT + (lax.iota(jnp.int32, L) & 3)) << RBITS)
    for j in range(K // L):
        psel[pl.ds(cnt + j * L, L)] = dumpv
    nch = (cnt + (K - 1)) // K
    lax.fori_loop(0, nch, flush, 0)

    for q in range(RPT // 80):
        @pl.when((wid < NW - 1) | (q == 0))
        def _copy():
            pltpu.sync_copy(acc.at[pl.ds(q * 80, 80)], out_hbm.at[pl.ds(base + q * 80, 80)])


_scatter_kernel = functools.partial(pl.kernel, **_SCAT_KW)(_scatter_body)


EB = 6400


def _pack_body(ei_ref, o_ref):
    v = ei_ref[...]
    o_ref[...] = ((v[1] << RBITS) | v[0])[None, :]


_pack = pl.pallas_call(
    _pack_body,
    grid=(E // EB,),
    in_specs=[pl.BlockSpec((2, EB), lambda i: (0, i))],
    out_specs=pl.BlockSpec((1, EB), lambda i: (0, i)),
    out_shape=jax.ShapeDtypeStruct((1, E), jnp.int32),
)


def _dinv_body(degs_ref, o_ref):
    deg = jnp.sum(degs_ref[...], axis=0) + 1.0
    o_ref[...] = lax.rsqrt(deg)[:, None]


_dinv = pl.pallas_call(
    _dinv_body,
    in_specs=[pl.BlockSpec((NW, N), lambda: (0, 0))],
    out_specs=pl.BlockSpec((N, 1), lambda: (0, 0)),
    out_shape=jax.ShapeDtypeStruct((N, 1), jnp.float32),
)

BR = 2000


def _mm_body(x_ref, w_ref, dinv_ref, o_ref):
    xw = jnp.dot(x_ref[...], w_ref[...], preferred_element_type=jnp.float32)
    o_ref[...] = xw * dinv_ref[...]


_mm = pl.pallas_call(
    _mm_body,
    grid=(N // BR,),
    in_specs=[
        pl.BlockSpec((BR, D), lambda i: (i, 0)),
        pl.BlockSpec((D, D), lambda i: (0, 0)),
        pl.BlockSpec((BR, 1), lambda i: (i, 0)),
    ],
    out_specs=pl.BlockSpec((BR, D), lambda i: (i, 0)),
    out_shape=jax.ShapeDtypeStruct((N, D), jnp.float32),
)

BR2 = 1000


def _ep_body(acc_ref, y_ref, dinv_ref, b_ref, o_ref):
    v = (acc_ref[...] + y_ref[...]) * dinv_ref[...] + b_ref[...]
    o_ref[...] = jnp.maximum(v, 0.0)


_ep = pl.pallas_call(
    _ep_body,
    grid=(N // BR2,),
    in_specs=[
        pl.BlockSpec((BR2, D), lambda i: (i, 0)),
        pl.BlockSpec((BR2, D), lambda i: (i, 0)),
        pl.BlockSpec((BR2, 1), lambda i: (i, 0)),
        pl.BlockSpec((1, D), lambda i: (0, 0)),
    ],
    out_specs=pl.BlockSpec((BR2, D), lambda i: (i, 0)),
    out_shape=jax.ShapeDtypeStruct((N, D), jnp.float32),
)


def kernel(x, edge_index, W, b):
    col = edge_index[1]
    pk = _pack(edge_index).reshape(E)
    degs = _deg_kernel(col)
    dinv = _dinv(degs)
    y = _mm(x, W, dinv)
    zin = jnp.zeros((K, D), jnp.float32)
    acc = _scatter_kernel(pk, y, zin)
    return _ep(acc, y, dinv, b.reshape(1, D))


# speedup vs baseline: 1.0440x; 1.0440x over previous
import functools

import jax
import jax.numpy as jnp
from jax import lax
from jax.experimental import pallas as pl
from jax.experimental.pallas import tpu as pltpu
from jax.experimental.pallas import tpu_sc as plsc

NC, NS, L = 2, 16, 16
NW = NC * NS

N = 10000
E = 160000
D = 256

K = 96
EPT_A = E // NW
RBITS = 14
RPT = 320
ACCR = 328
CH = 4000
SEL = 4224

_mesh = plsc.VectorSubcoreMesh(
    core_axis_name="c", subcore_axis_name="s", num_cores=NC, num_subcores=NS
)
_sc_params = pltpu.CompilerParams(needs_layout_passes=False)


_DEG_KW = dict(
    out_type=jax.ShapeDtypeStruct((NW, N), jnp.float32),
    mesh=_mesh,
    scratch_types=[
        pltpu.VMEM((EPT_A,), jnp.int32),
        pltpu.VMEM((N,), jnp.float32),
    ],
    compiler_params=_sc_params,
)


def _deg_body(col_hbm, dego_hbm, col_v, deg_v):
    c = lax.axis_index("c")
    s = lax.axis_index("s")
    wid = s * NC + c
    pltpu.sync_copy(col_hbm.at[pl.ds(wid * EPT_A, EPT_A)], col_v)

    def zero_body(i, carry):
        deg_v[pl.ds(i * L, L)] = jnp.zeros((L,), jnp.float32)
        return carry

    lax.fori_loop(0, N // L, zero_body, 0)

    ones = jnp.ones((L,), jnp.float32)
    nfull = EPT_A // L

    def add_body(i, carry):
        idx = col_v[pl.ds(i * L, L)]
        plsc.addupdate_scatter(deg_v, [idx], ones)
        return carry

    lax.fori_loop(0, nfull, add_body, 0)
    rem = EPT_A - nfull * L
    if rem:
        idx = col_v[pl.ds(EPT_A - L, L)]
        m = lax.iota(jnp.int32, L) >= (L - rem)
        plsc.addupdate_scatter(deg_v, [idx], ones, mask=m)
    pltpu.sync_copy(deg_v, dego_hbm.at[wid])


_deg_kernel = functools.partial(pl.kernel, **_DEG_KW)(_deg_body)


_SCAT_KW = dict(
    out_type=jax.ShapeDtypeStruct((N, D), jnp.float32),
    mesh=_mesh,
    scratch_types=[
        pltpu.VMEM((CH,), jnp.int32),
        pltpu.VMEM((CH,), jnp.int32),
        pltpu.VMEM((SEL,), jnp.int32),
        pltpu.VMEM((K,), jnp.int32),
        pltpu.VMEM((K,), jnp.int32),
        pltpu.VMEM((K, D), jnp.float32),
        pltpu.VMEM((ACCR, D), jnp.float32),
        pltpu.SemaphoreType.DMA,
        pltpu.SemaphoreType.DMA,
        pltpu.SemaphoreType.DMA,
        pltpu.SemaphoreType.DMA,
        pltpu.SemaphoreType.DMA,
    ],
    compiler_params=_sc_params,
)


def _scatter_body(pk_hbm, y_hbm, zin_hbm, out_hbm,
                  pk0, pk1, psel, rowstage, lstage, gbuf, acc,
                  gsem0, gsem1, gsem2, esem0, esem1):
    c = lax.axis_index("c")
    s = lax.axis_index("s")
    wid = s * NC + c
    base = wid * RPT

    for q in range(ACCR // K):
        pltpu.sync_copy(zin_hbm, acc.at[pl.ds(q * K, K)])
    remz = ACCR - (ACCR // K) * K
    pltpu.sync_copy(zin_hbm.at[pl.ds(0, remz)], acc.at[pl.ds((ACCR // K) * K, remz)])

    def flush(j, nothing):
        off = j * K
        for k8 in range(K // L):
            p = psel[pl.ds(off + k8 * L, L)]
            rowstage[pl.ds(k8 * L, L)] = p & ((1 << RBITS) - 1)
            lstage[pl.ds(k8 * L, L)] = lax.shift_right_logical(p, RBITS)
        pltpu.async_copy(y_hbm.at[rowstage], gbuf, gsem0).wait()

        @plsc.parallel_loop(0, K // L, unroll=2)
        def add_group(g):
            dvec = lstage[pl.ds(g * L, L)]
            rbase = g * L
            for i in range(L):
                d = dvec[i]
                for k in range(D // L):
                    plsc.addupdate(
                        acc.at[d, pl.ds(k * L, L)], gbuf[rbase + i, pl.ds(k * L, L)]
                    )

        return nothing

    lim = jnp.uint32(RPT << RBITS)
    boff = base << RBITS

    def scan_buf(buf, cnt):
        def comp_body(i, cnt):
            q = buf[pl.ds(i * L, L)] - boff
            m = plsc.bitcast(q, jnp.uint32) < lim
            plsc.store_compressed(psel.at[pl.ds(cnt, L)], q, mask=m)
            return cnt + jnp.sum(m.astype(jnp.int32))

        cnt = lax.fori_loop(0, CH // L, comp_body, cnt)
        nfl = cnt // K
        lax.fori_loop(0, nfl, flush, 0)
        for k8 in range(K // L):
            v = psel[pl.ds(nfl * K + k8 * L, L)]
            psel[pl.ds(k8 * L, L)] = v
        return cnt - nfl * K

    T = E // CH
    pltpu.async_copy(pk_hbm.at[pl.ds(0, CH)], pk0, esem0)

    def scan_pair(tt, cnt):
        t = 2 * tt
        d1 = pltpu.async_copy(pk_hbm.at[pl.ds((t + 1) * CH, CH)], pk1, esem1)
        pltpu.make_async_copy(pk_hbm.at[pl.ds(t * CH, CH)], pk0, esem0).wait()
        cnt = scan_buf(pk0, cnt)

        @pl.when(t + 2 < T)
        def _prefetch():
            pltpu.async_copy(pk_hbm.at[pl.ds((t + 2) * CH, CH)], pk0, esem0)

        d1.wait()
        cnt = scan_buf(pk1, cnt)
        return cnt

    cnt = lax.fori_loop(0, T // 2, scan_pair, jnp.int32(0))

    dumpv = ((RPT + (lax.iota(jnp.int32, L) & 3)) << RBITS)
    for j in range(K // L):
        psel[pl.ds(cnt + j * L, L)] = dumpv
    nch = (cnt + (K - 1)) // K
    lax.fori_loop(0, nch, flush, 0)

    for q in range(RPT // 80):
        @pl.when((wid < NW - 1) | (q == 0))
        def _copy():
            pltpu.sync_copy(acc.at[pl.ds(q * 80, 80)], out_hbm.at[pl.ds(base + q * 80, 80)])


_scatter_kernel = functools.partial(pl.kernel, **_SCAT_KW)(_scatter_body)


EB = 6400


def _pack_body(ei_ref, o_ref):
    v = ei_ref[...]
    o_ref[...] = ((v[1] << RBITS) | v[0])[None, :]


_pack = pl.pallas_call(
    _pack_body,
    grid=(E // EB,),
    in_specs=[pl.BlockSpec((2, EB), lambda i: (0, i))],
    out_specs=pl.BlockSpec((1, EB), lambda i: (0, i)),
    out_shape=jax.ShapeDtypeStruct((1, E), jnp.int32),
)


def _dinv_body(degs_ref, o_ref):
    deg = jnp.sum(degs_ref[...], axis=0) + 1.0
    o_ref[...] = lax.rsqrt(deg)[:, None]


_dinv = pl.pallas_call(
    _dinv_body,
    in_specs=[pl.BlockSpec((NW, N), lambda: (0, 0))],
    out_specs=pl.BlockSpec((N, 1), lambda: (0, 0)),
    out_shape=jax.ShapeDtypeStruct((N, 1), jnp.float32),
)

BR = 2000


def _mm_body(x_ref, w_ref, dinv_ref, o_ref):
    xw = jnp.dot(x_ref[...], w_ref[...], preferred_element_type=jnp.float32)
    o_ref[...] = xw * dinv_ref[...]


_mm = pl.pallas_call(
    _mm_body,
    grid=(N // BR,),
    in_specs=[
        pl.BlockSpec((BR, D), lambda i: (i, 0)),
        pl.BlockSpec((D, D), lambda i: (0, 0)),
        pl.BlockSpec((BR, 1), lambda i: (i, 0)),
    ],
    out_specs=pl.BlockSpec((BR, D), lambda i: (i, 0)),
    out_shape=jax.ShapeDtypeStruct((N, D), jnp.float32),
)

BR2 = 1000


def _ep_body(acc_ref, y_ref, dinv_ref, b_ref, o_ref):
    v = (acc_ref[...] + y_ref[...]) * dinv_ref[...] + b_ref[...]
    o_ref[...] = jnp.maximum(v, 0.0)


_ep = pl.pallas_call(
    _ep_body,
    grid=(N // BR2,),
    in_specs=[
        pl.BlockSpec((BR2, D), lambda i: (i, 0)),
        pl.BlockSpec((BR2, D), lambda i: (i, 0)),
        pl.BlockSpec((BR2, 1), lambda i: (i, 0)),
        pl.BlockSpec((1, D), lambda i: (0, 0)),
    ],
    out_specs=pl.BlockSpec((BR2, D), lambda i: (i, 0)),
    out_shape=jax.ShapeDtypeStruct((N, D), jnp.float32),
)


def kernel(x, edge_index, W, b):
    col = edge_index[1]
    pk = _pack(edge_index).reshape(E)
    degs = _deg_kernel(col)
    dinv = _dinv(degs)
    y = _mm(x, W, dinv)
    zin = jnp.zeros((K, D), jnp.float32)
    acc = _scatter_kernel(pk, y, zin)
    return _ep(acc, y, dinv, b.reshape(1, D))
